# Initial kernel scaffold; baseline (speedup 1.0000x reference)
import jax
import jax.numpy as jnp
from jax.experimental import pallas as pl


def kernel(x, edge_index, dst_node_ids, W1, b1, W2, b2):
    raise NotImplementedError("write your pallas kernel here")


# speedup vs baseline: 8.6064x; 8.6064x over previous
import functools

import jax
import jax.numpy as jnp
from jax import lax
from jax.experimental import pallas as pl
from jax.experimental.pallas import tpu as pltpu
import jax.experimental.pallas.tpu_sc as plsc

N_NODES = 10000
DIM = 128
N_EDGES = 320000

NC = 2
NS = 16
NW = NC * NS
K = 128
C = (N_EDGES + NW * K - 1) // (NW * K)
E_PAD = NW * C * K
N_PAD = 10240
RPT = N_PAD // NS


def _sc_mesh():
    return plsc.VectorSubcoreMesh(
        core_axis_name="c", subcore_axis_name="s", num_cores=NC,
        num_subcores=NS)


def _deg_body(zeros_hbm, ones_hbm, dst_hbm, out_hbm, dstv, onev, accd):
    c = lax.axis_index("c")
    s = lax.axis_index("s")
    wid = s * NC + c
    pltpu.sync_copy(zeros_hbm.at[pl.ds(s * RPT, RPT)],
                    accd.at[pl.ds(s * RPT, RPT)])
    pltpu.sync_copy(ones_hbm, onev)
    plsc.subcore_barrier()

    def body(j, carry):
        row = wid * C + j
        pltpu.sync_copy(dst_hbm.at[row], dstv)
        pltpu.sync_copy(onev, accd.at[dstv], add=True)
        return carry

    lax.fori_loop(0, C, body, 0)
    plsc.subcore_barrier()
    pltpu.sync_copy(accd.at[pl.ds(s * RPT, RPT)],
                    out_hbm.at[pl.ds(c * N_PAD + s * RPT, RPT)])


@jax.jit
def _sc_degree(zeros_n, ones_k, dstp):
    return pl.kernel(
        _deg_body,
        out_type=jax.ShapeDtypeStruct((NC * N_PAD,), jnp.float32),
        mesh=_sc_mesh(),
        scratch_types=[
            pltpu.VMEM((K,), jnp.int32),
            pltpu.VMEM((K,), jnp.float32),
            pltpu.VMEM_SHARED((N_PAD,), jnp.float32),
        ],
    )(zeros_n, ones_k, dstp)


def _scatter_body(table_hbm, src_hbm, dst_hbm, out_hbm,
                  srcv, dstv, rows, acc, gsem):
    c = lax.axis_index("c")
    s = lax.axis_index("s")
    wid = s * NC + c
    pltpu.sync_copy(table_hbm.at[pl.ds(s * RPT, RPT)],
                    acc.at[pl.ds(s * RPT, RPT)])
    plsc.subcore_barrier()

    def body(j, carry):
        row = wid * C + j
        pltpu.sync_copy(src_hbm.at[row], srcv)
        pltpu.sync_copy(dst_hbm.at[row], dstv)
        pltpu.async_copy(table_hbm.at[srcv], rows, gsem).wait()
        pltpu.sync_copy(rows, acc.at[dstv], add=True)
        return carry

    lax.fori_loop(0, C, body, 0)
    plsc.subcore_barrier()
    pltpu.sync_copy(acc.at[pl.ds(s * RPT, RPT)],
                    out_hbm.at[pl.ds(c * N_PAD + s * RPT, RPT)])


@jax.jit
def _sc_scatter(table, srcp, dstp):
    return pl.kernel(
        _scatter_body,
        out_type=jax.ShapeDtypeStruct((NC * N_PAD, DIM), jnp.float32),
        mesh=_sc_mesh(),
        scratch_types=[
            pltpu.VMEM((K,), jnp.int32),
            pltpu.VMEM((K,), jnp.int32),
            pltpu.VMEM((K, DIM), jnp.float32),
            pltpu.VMEM_SHARED((N_PAD, DIM), jnp.float32),
            pltpu.SemaphoreType.DMA,
        ],
    )(table, srcp, dstp)


BN = 1024


def _tc_norm_mm_body(deg_ref, x_ref, w_ref, norm_ref, hw_ref):
    deg = deg_ref[0, :] + deg_ref[1, :] + 1.0
    nrm = lax.rsqrt(deg)
    norm_ref[...] = nrm
    hw_ref[...] = jnp.dot(x_ref[...] * nrm[:, None], w_ref[...],
                          preferred_element_type=jnp.float32)


@jax.jit
def _tc_norm_mm(degp, x, w):
    grid = (N_PAD // BN,)
    return pl.pallas_call(
        _tc_norm_mm_body,
        grid=grid,
        in_specs=[
            pl.BlockSpec((2, BN), lambda i: (0, i)),
            pl.BlockSpec((BN, DIM), lambda i: (i, 0)),
            pl.BlockSpec((DIM, DIM), lambda i: (0, 0)),
        ],
        out_specs=[
            pl.BlockSpec((BN,), lambda i: (i,)),
            pl.BlockSpec((BN, DIM), lambda i: (i, 0)),
        ],
        out_shape=[
            jax.ShapeDtypeStruct((N_PAD,), jnp.float32),
            jax.ShapeDtypeStruct((N_PAD, DIM), jnp.float32),
        ],
    )(degp, x, w)


def _tc_mid_body(p_ref, hw_ref, norm_ref, b_ref, w_ref, out_ref):
    agg = p_ref[0] + p_ref[1] - hw_ref[...]
    nrm = norm_ref[...]
    h = agg * nrm[:, None] + b_ref[...]
    out_ref[...] = jnp.dot(h * nrm[:, None], w_ref[...],
                           preferred_element_type=jnp.float32)


@jax.jit
def _tc_mid(p, hw, norm, b, w):
    grid = (N_PAD // BN,)
    return pl.pallas_call(
        _tc_mid_body,
        grid=grid,
        in_specs=[
            pl.BlockSpec((2, BN, DIM), lambda i: (0, i, 0)),
            pl.BlockSpec((BN, DIM), lambda i: (i, 0)),
            pl.BlockSpec((BN,), lambda i: (i,)),
            pl.BlockSpec((1, DIM), lambda i: (0, 0)),
            pl.BlockSpec((DIM, DIM), lambda i: (0, 0)),
        ],
        out_specs=pl.BlockSpec((BN, DIM), lambda i: (i, 0)),
        out_shape=jax.ShapeDtypeStruct((N_PAD, DIM), jnp.float32),
    )(p, hw, norm, b, w)


def _tc_final_body(p_ref, hw_ref, norm_ref, b_ref, out_ref):
    agg = p_ref[0] + p_ref[1] - hw_ref[...]
    nrm = norm_ref[...]
    out_ref[...] = agg * nrm[:, None] + b_ref[...]


@jax.jit
def _tc_final(p, hw, norm, b):
    grid = (N_PAD // BN,)
    return pl.pallas_call(
        _tc_final_body,
        grid=grid,
        in_specs=[
            pl.BlockSpec((2, BN, DIM), lambda i: (0, i, 0)),
            pl.BlockSpec((BN, DIM), lambda i: (i, 0)),
            pl.BlockSpec((BN,), lambda i: (i,)),
            pl.BlockSpec((1, DIM), lambda i: (0, 0)),
        ],
        out_specs=pl.BlockSpec((BN, DIM), lambda i: (i, 0)),
        out_shape=jax.ShapeDtypeStruct((N_PAD, DIM), jnp.float32),
    )(p, hw, norm, b)


def kernel(x, edge_index, dst_node_ids, W1, b1, W2, b2):
    src = edge_index[0]
    dst = edge_index[1]
    pad_n = E_PAD - N_EDGES
    pad_idx = (N_NODES
               + (jnp.arange(pad_n, dtype=jnp.int32) % (N_PAD - N_NODES)))
    srcp = jnp.concatenate([src, pad_idx]).reshape(NW * C, K)
    dstp = jnp.concatenate([dst, pad_idx]).reshape(NW * C, K)
    x_pad = jnp.zeros((N_PAD, DIM), jnp.float32).at[:N_NODES].set(x)

    zeros_n = jnp.zeros((N_PAD,), jnp.float32)
    ones_k = jnp.ones((K,), jnp.float32)

    degp = _sc_degree(zeros_n, ones_k, dstp).reshape(NC, N_PAD)
    norm, hw1 = _tc_norm_mm(degp, x_pad, W1)
    p1 = _sc_scatter(hw1, srcp, dstp).reshape(NC, N_PAD, DIM)
    hw2 = _tc_mid(p1, hw1, norm, b1.reshape(1, DIM), W2)
    p2 = _sc_scatter(hw2, srcp, dstp).reshape(NC, N_PAD, DIM)
    h2 = _tc_final(p2, hw2, norm, b2.reshape(1, DIM))
    return (h2[:N_NODES], dst_node_ids)


# speedup vs baseline: 15.4018x; 1.7896x over previous
import functools

import jax
import jax.numpy as jnp
from jax import lax
from jax.experimental import pallas as pl
from jax.experimental.pallas import tpu as pltpu
import jax.experimental.pallas.tpu_sc as plsc

N_NODES = 10000
DIM = 128
N_EDGES = 320000

NC = 2
NS = 16
NW = NC * NS
K = 128
C = 80
E_PAD = NW * C * K
N_PAD = 10240
RPT = N_PAD // NS
PCH = 2


def _sc_mesh():
    return plsc.VectorSubcoreMesh(
        core_axis_name="c", subcore_axis_name="s", num_cores=NC,
        num_subcores=NS)


_DEG_FIRE = 8


def _deg_body(zeros_hbm, ones_hbm, dst_hbm, out_hbm, dstb, onev, accd, sem):
    c = lax.axis_index("c")
    s = lax.axis_index("s")
    wid = s * NC + c
    pltpu.sync_copy(dst_hbm.at[wid], dstb)
    pltpu.sync_copy(ones_hbm, onev)
    pltpu.sync_copy(zeros_hbm.at[pl.ds(s * RPT, RPT)],
                    accd.at[pl.ds(s * RPT, RPT)])
    plsc.subcore_barrier()

    def body(g, carry):
        for b in range(_DEG_FIRE):
            pltpu.async_copy(onev, accd.at[dstb.at[g * _DEG_FIRE + b]], sem,
                             add=True)
        for b in range(_DEG_FIRE):
            pltpu.make_async_copy(onev, accd.at[dstb.at[g * _DEG_FIRE + b]],
                                  sem).wait()
        return carry

    lax.fori_loop(0, C // _DEG_FIRE, body, 0)
    plsc.subcore_barrier()
    pltpu.sync_copy(accd.at[pl.ds(s * RPT, RPT)],
                    out_hbm.at[pl.ds(c * N_PAD + s * RPT, RPT)])


@jax.jit
def _sc_degree(zeros_n, ones_k, dstp):
    return pl.kernel(
        _deg_body,
        out_type=jax.ShapeDtypeStruct((NC * N_PAD,), jnp.float32),
        mesh=_sc_mesh(),
        scratch_types=[
            pltpu.VMEM((C + PCH, K), jnp.int32),
            pltpu.VMEM((K,), jnp.float32),
            pltpu.VMEM_SHARED((N_PAD,), jnp.float32),
            pltpu.SemaphoreType.DMA,
        ],
    )(zeros_n, ones_k, dstp)


def _scatter_body(table_hbm, src_hbm, dst_hbm, out_hbm,
                  srcv, dstv, rows, acc, gsem, isem):
    c = lax.axis_index("c")
    s = lax.axis_index("s")
    wid = s * NC + c

    def idx_start(j, par):
        pltpu.async_copy(src_hbm.at[wid, j], srcv.at[par], isem)
        pltpu.async_copy(dst_hbm.at[wid, j], dstv.at[par], isem)

    def idx_wait(j, par):
        pltpu.make_async_copy(src_hbm.at[wid, j], srcv.at[par], isem).wait()
        pltpu.make_async_copy(dst_hbm.at[wid, j], dstv.at[par], isem).wait()

    idx_start(0, 0)
    pltpu.sync_copy(table_hbm.at[pl.ds(s * RPT, RPT)],
                    acc.at[pl.ds(s * RPT, RPT)])
    plsc.subcore_barrier()

    idx_wait(0, 0)
    pltpu.async_copy(table_hbm.at[srcv.at[0]], rows.at[0], gsem)
    idx_start(1, 1)

    def body(jj, carry):
        for par in (0, 1):
            j = jj * 2 + par
            npar = 1 - par
            pltpu.make_async_copy(table_hbm.at[srcv.at[par]],
                                  rows.at[par], gsem).wait()
            idx_wait(j + 1, npar)
            pltpu.async_copy(table_hbm.at[srcv.at[npar]],
                             rows.at[npar], gsem)
            pltpu.sync_copy(rows.at[par], acc.at[dstv.at[par]], add=True)
            idx_start(j + 2, par)
        return carry

    lax.fori_loop(0, C // 2, body, 0)
    pltpu.make_async_copy(table_hbm.at[srcv.at[0]], rows.at[0], gsem).wait()
    idx_wait(C + 1, 1)
    plsc.subcore_barrier()
    pltpu.sync_copy(acc.at[pl.ds(s * RPT, RPT)],
                    out_hbm.at[pl.ds(c * N_PAD + s * RPT, RPT)])


@jax.jit
def _sc_scatter(table, srcp, dstp):
    return pl.kernel(
        _scatter_body,
        out_type=jax.ShapeDtypeStruct((NC * N_PAD, DIM), jnp.float32),
        mesh=_sc_mesh(),
        scratch_types=[
            pltpu.VMEM((2, K), jnp.int32),
            pltpu.VMEM((2, K), jnp.int32),
            pltpu.VMEM((2, K, DIM), jnp.float32),
            pltpu.VMEM_SHARED((N_PAD, DIM), jnp.float32),
            pltpu.SemaphoreType.DMA,
            pltpu.SemaphoreType.DMA,
        ],
    )(table, srcp, dstp)


BN = 1024


def _tc_norm_mm_body(deg_ref, x_ref, w_ref, norm_ref, hw_ref):
    deg = deg_ref[0, :] + deg_ref[1, :] + 1.0
    nrm = lax.rsqrt(deg)
    norm_ref[...] = nrm
    hw_ref[...] = jnp.dot(x_ref[...] * nrm[:, None], w_ref[...],
                          preferred_element_type=jnp.float32)


@jax.jit
def _tc_norm_mm(degp, x, w):
    grid = (N_PAD // BN,)
    return pl.pallas_call(
        _tc_norm_mm_body,
        grid=grid,
        in_specs=[
            pl.BlockSpec((2, BN), lambda i: (0, i)),
            pl.BlockSpec((BN, DIM), lambda i: (i, 0)),
            pl.BlockSpec((DIM, DIM), lambda i: (0, 0)),
        ],
        out_specs=[
            pl.BlockSpec((BN,), lambda i: (i,)),
            pl.BlockSpec((BN, DIM), lambda i: (i, 0)),
        ],
        out_shape=[
            jax.ShapeDtypeStruct((N_PAD,), jnp.float32),
            jax.ShapeDtypeStruct((N_PAD, DIM), jnp.float32),
        ],
    )(degp, x, w)


def _tc_mid_body(p_ref, hw_ref, norm_ref, b_ref, w_ref, out_ref):
    agg = p_ref[0] + p_ref[1] - hw_ref[...]
    nrm = norm_ref[...]
    h = agg * nrm[:, None] + b_ref[...]
    out_ref[...] = jnp.dot(h * nrm[:, None], w_ref[...],
                           preferred_element_type=jnp.float32)


@jax.jit
def _tc_mid(p, hw, norm, b, w):
    grid = (N_PAD // BN,)
    return pl.pallas_call(
        _tc_mid_body,
        grid=grid,
        in_specs=[
            pl.BlockSpec((2, BN, DIM), lambda i: (0, i, 0)),
            pl.BlockSpec((BN, DIM), lambda i: (i, 0)),
            pl.BlockSpec((BN,), lambda i: (i,)),
            pl.BlockSpec((1, DIM), lambda i: (0, 0)),
            pl.BlockSpec((DIM, DIM), lambda i: (0, 0)),
        ],
        out_specs=pl.BlockSpec((BN, DIM), lambda i: (i, 0)),
        out_shape=jax.ShapeDtypeStruct((N_PAD, DIM), jnp.float32),
    )(p, hw, norm, b, w)


def _tc_final_body(p_ref, hw_ref, norm_ref, b_ref, out_ref):
    agg = p_ref[0] + p_ref[1] - hw_ref[...]
    nrm = norm_ref[...]
    out_ref[...] = agg * nrm[:, None] + b_ref[...]


@jax.jit
def _tc_final(p, hw, norm, b):
    grid = (N_PAD // BN,)
    return pl.pallas_call(
        _tc_final_body,
        grid=grid,
        in_specs=[
            pl.BlockSpec((2, BN, DIM), lambda i: (0, i, 0)),
            pl.BlockSpec((BN, DIM), lambda i: (i, 0)),
            pl.BlockSpec((BN,), lambda i: (i,)),
            pl.BlockSpec((1, DIM), lambda i: (0, 0)),
        ],
        out_specs=pl.BlockSpec((BN, DIM), lambda i: (i, 0)),
        out_shape=jax.ShapeDtypeStruct((N_PAD, DIM), jnp.float32),
    )(p, hw, norm, b)


def kernel(x, edge_index, dst_node_ids, W1, b1, W2, b2):
    src = edge_index[0]
    dst = edge_index[1]
    pad_n = E_PAD - N_EDGES
    pad_idx = (N_NODES
               + (jnp.arange(pad_n, dtype=jnp.int32) % (N_PAD - N_NODES)))
    extra = (N_NODES + (jnp.arange(NW * PCH * K, dtype=jnp.int32)
                        % (N_PAD - N_NODES))).reshape(NW, PCH, K)
    srcp = jnp.concatenate(
        [jnp.concatenate([src, pad_idx]).reshape(NW, C, K), extra], axis=1)
    dstp = jnp.concatenate(
        [jnp.concatenate([dst, pad_idx]).reshape(NW, C, K), extra], axis=1)
    x_pad = jnp.zeros((N_PAD, DIM), jnp.float32).at[:N_NODES].set(x)

    zeros_n = jnp.zeros((N_PAD,), jnp.float32)
    ones_k = jnp.ones((K,), jnp.float32)

    degp = _sc_degree(zeros_n, ones_k, dstp).reshape(NC, N_PAD)
    norm, hw1 = _tc_norm_mm(degp, x_pad, W1)
    p1 = _sc_scatter(hw1, srcp, dstp).reshape(NC, N_PAD, DIM)
    hw2 = _tc_mid(p1, hw1, norm, b1.reshape(1, DIM), W2)
    p2 = _sc_scatter(hw2, srcp, dstp).reshape(NC, N_PAD, DIM)
    h2 = _tc_final(p2, hw2, norm, b2.reshape(1, DIM))
    return (h2[:N_NODES], dst_node_ids)
